# baseline (speedup 1.0000x reference)
import functools

import jax
import jax.numpy as jnp
from jax import lax
from jax.experimental import pallas as pl
from jax.experimental.pallas import tpu as pltpu
from jax.experimental.pallas import tpu_sc as plsc

N = 10000
NH = N // 2
F = 256
HID = 256
CLS = 64
E = 160000

NC = 2
NS = 16
LANES = 16

EDGES_PER_SUB = E // NS
CHUNK = 80
NCHUNK = EDGES_PER_SUB // CHUNK
NBUF = 5
N_PAD = 10112
ROWS_PER_SUB = N_PAD // NS

DEG_PER_W = E // (NC * NS)
_DEG_FULL = DEG_PER_W // LANES
_DEG_TAIL = DEG_PER_W - _DEG_FULL * LANES

_sc_mesh = plsc.VectorSubcoreMesh(core_axis_name="c", subcore_axis_name="s")


@functools.partial(
    pl.kernel,
    out_type=jax.ShapeDtypeStruct((NC * NS, N), jnp.float32),
    mesh=_sc_mesh,
    scratch_types=[
        pltpu.VMEM((DEG_PER_W + LANES,), jnp.int32),
        pltpu.VMEM((N,), jnp.float32),
    ],
    compiler_params=pltpu.CompilerParams(needs_layout_passes=False,
                                         use_tc_tiling_on_sc=False),
)
def _deg_kernel(dst_hbm, out_hbm, idx_v, hist_v):
    c = lax.axis_index("c")
    s = lax.axis_index("s")
    wid = s * NC + c
    base = wid * DEG_PER_W
    idx_v[pl.ds(_DEG_FULL * LANES, LANES)] = jnp.zeros((LANES,), jnp.int32)
    pltpu.sync_copy(dst_hbm.at[pl.ds(base, DEG_PER_W)],
                    idx_v.at[pl.ds(0, DEG_PER_W)])

    @pl.loop(0, N // LANES)
    def _zero(i):
        hist_v[pl.ds(i * LANES, LANES)] = jnp.zeros((LANES,), jnp.float32)

    ones = jnp.ones((LANES,), jnp.float32)

    @pl.loop(0, _DEG_FULL)
    def _acc(i):
        idx = idx_v[pl.ds(i * LANES, LANES)]
        plsc.addupdate_scatter(hist_v, [idx], ones)

    tidx = idx_v[pl.ds(_DEG_FULL * LANES, LANES)]
    tmask = lax.iota(jnp.int32, LANES) < _DEG_TAIL
    plsc.addupdate_scatter(hist_v, [tidx], ones, mask=tmask)
    pltpu.sync_copy(hist_v, out_hbm.at[wid])


def _make_scatter(width, npass):
    NG = NCHUNK // NBUF
    assert NG % 2 == 1 and NG >= 5

    @functools.partial(
        pl.kernel,
        out_type=jax.ShapeDtypeStruct((npass, N_PAD, 128), jnp.float32),
        mesh=_sc_mesh,
        scratch_types=[
            pltpu.VMEM((NCHUNK, CHUNK), jnp.int32),
            pltpu.VMEM((NCHUNK, CHUNK), jnp.int32),
            pltpu.VMEM((2 * NBUF, CHUNK, width), jnp.float32),
            pltpu.VMEM_SHARED((N_PAD, width), jnp.float32),
            pltpu.SemaphoreType.DMA((2 * NBUF,)),
            pltpu.SemaphoreType.DMA((2 * NBUF,)),
        ],
        compiler_params=pltpu.CompilerParams(needs_layout_passes=False,
                                             use_tc_tiling_on_sc=False),
    )
    def _scatter(hs_hbm, srcq_hbm, dst_hbm, zeros_hbm, out_hbm,
                 src_v, dst_v, rows_v, acc, gsem, ssem):
        c = lax.axis_index("c")
        s = lax.axis_index("s")
        slab = pl.ds(s * ROWS_PER_SUB, ROWS_PER_SUB)
        pltpu.sync_copy(dst_hbm.at[s], dst_v)

        def gather(k, slot):
            pltpu.async_copy(hs_hbm.at[src_v.at[k]], rows_v.at[slot],
                             gsem.at[slot])

        def do_group(g, bank):
            descs = []
            for b in range(NBUF):
                slot = bank * NBUF + b
                k = g * NBUF + b
                pltpu.make_async_copy(hs_hbm.at[src_v.at[k]], rows_v.at[slot],
                                      gsem.at[slot]).wait()
                descs.append(pltpu.async_copy(rows_v.at[slot],
                                              acc.at[dst_v.at[k]],
                                              ssem.at[slot], add=True))
            return descs

        def regather(g, bank, descs):
            for b in range(NBUF):
                descs[b].wait()
                gather(g * NBUF + b, bank * NBUF + b)

        @pl.loop(0, npass)
        def _pass(p):
            pltpu.sync_copy(srcq_hbm.at[p * NC + c, s], src_v)
            pltpu.sync_copy(zeros_hbm.at[slab], acc.at[slab])
            plsc.subcore_barrier()

            for slot in range(2 * NBUF):
                gather(slot, slot)

            @pl.loop(0, (NG - 3) // 2)
            def _g2(g2):
                for x in range(2):
                    g = 2 * g2 + x
                    regather(g + 2, x, do_group(g, x))

            d0 = do_group(NG - 3, (NG - 3) % 2)
            regather(NG - 1, (NG - 3) % 2, d0)
            tail = (do_group(NG - 2, (NG - 2) % 2)
                    + do_group(NG - 1, (NG - 1) % 2))
            for d in tail:
                d.wait()

            plsc.subcore_barrier()
            pltpu.sync_copy(acc.at[slab],
                            out_hbm.at[p].at[slab, pl.ds(c * width, width)])

    return _scatter


_scatter_hid = _make_scatter(HID // 4, 2)
_scatter_cls = _make_scatter(CLS // 2, 1)


_NBH = 1000
_GRID = NH // _NBH


def _dinv_of(dp_block):
    deg = jnp.sum(dp_block, axis=1) + 1.0
    return lax.rsqrt(deg)


def _mm1_body(xe_ref, xo_ref, w_ref, dpe_ref, dpo_ref, o4_ref):
    q = HID // 4
    dinve = _dinv_of(dpe_ref[...])
    dinvo = _dinv_of(dpo_ref[...])
    hse = jnp.dot(xe_ref[...], w_ref[...],
                  preferred_element_type=jnp.float32) * dinve[:, None]
    hso = jnp.dot(xo_ref[...], w_ref[...],
                  preferred_element_type=jnp.float32) * dinvo[:, None]
    for j in range(4):
        o4_ref[j] = jnp.concatenate(
            [hse[:, j * q:(j + 1) * q], hso[:, j * q:(j + 1) * q]], axis=1)


_mm1 = pl.pallas_call(
    _mm1_body,
    grid=(_GRID,),
    in_specs=[pl.BlockSpec((_NBH, F), lambda i: (i, 0)),
              pl.BlockSpec((_NBH, F), lambda i: (_GRID + i, 0)),
              pl.BlockSpec((F, HID), lambda i: (0, 0)),
              pl.BlockSpec((_NBH, NC * NS), lambda i: (i, 0)),
              pl.BlockSpec((_NBH, NC * NS), lambda i: (_GRID + i, 0))],
    out_specs=pl.BlockSpec((4, _NBH, 128), lambda i: (0, i, 0)),
    out_shape=jax.ShapeDtypeStruct((4, NH, 128), jnp.float32),
)


def _mid_body(ae_ref, ao_ref, t1_ref, dpe_ref, dpo_ref, b_ref, w_ref, o_ref):
    q = HID // 4
    dinve = _dinv_of(dpe_ref[...])
    dinvo = _dinv_of(dpo_ref[...])

    def half(a_ref, dinv, lo, hi):
        agg = jnp.concatenate([a_ref[0], a_ref[1]], axis=1)
        hs = jnp.concatenate([t1_ref[j][:, lo:hi] for j in range(4)], axis=1)
        z = (agg + hs) * dinv[:, None] + b_ref[...]
        a = jnp.maximum(z, 0.0)
        h2 = jnp.dot(a, w_ref[...], preferred_element_type=jnp.float32)
        return h2 * dinv[:, None]

    t2e = half(ae_ref, dinve, 0, q)
    t2o = half(ao_ref, dinvo, q, 2 * q)
    o_ref[...] = jnp.concatenate([t2e, t2o], axis=1)


_mid = pl.pallas_call(
    _mid_body,
    grid=(_GRID,),
    in_specs=[pl.BlockSpec((2, _NBH, 128), lambda i: (0, i, 0)),
              pl.BlockSpec((2, _NBH, 128), lambda i: (0, _GRID + i, 0)),
              pl.BlockSpec((4, _NBH, 128), lambda i: (0, i, 0)),
              pl.BlockSpec((_NBH, NC * NS), lambda i: (i, 0)),
              pl.BlockSpec((_NBH, NC * NS), lambda i: (_GRID + i, 0)),
              pl.BlockSpec((1, HID), lambda i: (0, 0)),
              pl.BlockSpec((HID, CLS), lambda i: (0, 0))],
    out_specs=pl.BlockSpec((_NBH, 128), lambda i: (i, 0)),
    out_shape=jax.ShapeDtypeStruct((NH, 128), jnp.float32),
)


def _final_body(ae_ref, ao_ref, t2_ref, dpe_ref, dpo_ref, b_ref, o_ref):
    dinve = _dinv_of(dpe_ref[...])
    dinvo = _dinv_of(dpo_ref[...])

    def half(a_ref, dinv, lo):
        z = ((a_ref[0][:, :CLS] + t2_ref[:, lo:lo + CLS])
             * dinv[:, None] + b_ref[...])
        m = jnp.max(z, axis=1, keepdims=True)
        lse = jnp.log(jnp.sum(jnp.exp(z - m), axis=1, keepdims=True)) + m
        return z - lse

    oute = half(ae_ref, dinve, 0)
    outo = half(ao_ref, dinvo, CLS)
    o_ref[0] = oute
    o_ref[1] = outo


_final = pl.pallas_call(
    _final_body,
    grid=(_GRID,),
    in_specs=[pl.BlockSpec((1, _NBH, 128), lambda i: (0, i, 0)),
              pl.BlockSpec((1, _NBH, 128), lambda i: (0, _GRID + i, 0)),
              pl.BlockSpec((_NBH, 128), lambda i: (i, 0)),
              pl.BlockSpec((_NBH, NC * NS), lambda i: (i, 0)),
              pl.BlockSpec((_NBH, NC * NS), lambda i: (_GRID + i, 0)),
              pl.BlockSpec((1, CLS), lambda i: (0, 0))],
    out_specs=pl.BlockSpec((2, _NBH, CLS), lambda i: (0, i, 0)),
    out_shape=jax.ShapeDtypeStruct((2, NH, CLS), jnp.float32),
)


def kernel(x, edge_index, W1, b1, W2, b2):
    src = edge_index[0].astype(jnp.int32)
    dst = edge_index[1].astype(jnp.int32)

    deg_parts = _deg_kernel(dst)
    dpt = deg_parts.T

    table1 = _mm1(x, x, W1, dpt, dpt)

    dst3 = dst.reshape(NS, NCHUNK, CHUNK)
    sh = (src >= NH).astype(jnp.int32)
    base1 = 2 * src - sh * (2 * NH - 1)
    srcq1 = (base1[None, :]
             + (jnp.arange(4, dtype=jnp.int32) * N)[:, None]
             ).reshape(4, NS, NCHUNK, CHUNK)
    srcq2 = (2 * base1[None, :]
             + jnp.arange(2, dtype=jnp.int32)[:, None]
             ).reshape(2, NS, NCHUNK, CHUNK)

    agg1 = _scatter_hid(table1.reshape(4 * N, HID // 4), srcq1, dst3,
                        jnp.zeros((N_PAD, HID // 4), jnp.float32))
    table2 = _mid(agg1, agg1, table1, dpt, dpt, b1.reshape(1, HID), W2)
    agg2 = _scatter_cls(table2.reshape(2 * N, CLS // 2), srcq2, dst3,
                        jnp.zeros((N_PAD, CLS // 2), jnp.float32))
    outp = _final(agg2, agg2, table2, dpt, dpt, b2.reshape(1, CLS))
    return outp.reshape(N, CLS)


# speedup vs baseline: 1.0339x; 1.0339x over previous
import functools

import jax
import jax.numpy as jnp
from jax import lax
from jax.experimental import pallas as pl
from jax.experimental.pallas import tpu as pltpu
from jax.experimental.pallas import tpu_sc as plsc

N = 10000
NH = N // 2
F = 256
HID = 256
CLS = 64
E = 160000

NC = 2
NS = 16
LANES = 16

EDGES_PER_SUB = E // NS
CHUNK = 80
NCHUNK = EDGES_PER_SUB // CHUNK
NBUF = 5
N_PAD = 10112
ROWS_PER_SUB = N_PAD // NS

DEG_PER_W = E // (NC * NS)
_DEG_FULL = DEG_PER_W // LANES
_DEG_TAIL = DEG_PER_W - _DEG_FULL * LANES

_sc_mesh = plsc.VectorSubcoreMesh(core_axis_name="c", subcore_axis_name="s")


@functools.partial(
    pl.kernel,
    out_type=jax.ShapeDtypeStruct((NC * NS, N), jnp.float32),
    mesh=_sc_mesh,
    scratch_types=[
        pltpu.VMEM((DEG_PER_W + LANES,), jnp.int32),
        pltpu.VMEM((N,), jnp.float32),
    ],
    compiler_params=pltpu.CompilerParams(needs_layout_passes=False,
                                         use_tc_tiling_on_sc=False),
)
def _deg_kernel(dst_hbm, out_hbm, idx_v, hist_v):
    c = lax.axis_index("c")
    s = lax.axis_index("s")
    wid = s * NC + c
    base = wid * DEG_PER_W
    idx_v[pl.ds(_DEG_FULL * LANES, LANES)] = jnp.zeros((LANES,), jnp.int32)
    pltpu.sync_copy(dst_hbm.at[pl.ds(base, DEG_PER_W)],
                    idx_v.at[pl.ds(0, DEG_PER_W)])

    @pl.loop(0, N // LANES)
    def _zero(i):
        hist_v[pl.ds(i * LANES, LANES)] = jnp.zeros((LANES,), jnp.float32)

    ones = jnp.ones((LANES,), jnp.float32)

    @pl.loop(0, _DEG_FULL)
    def _acc(i):
        idx = idx_v[pl.ds(i * LANES, LANES)]
        plsc.addupdate_scatter(hist_v, [idx], ones)

    tidx = idx_v[pl.ds(_DEG_FULL * LANES, LANES)]
    tmask = lax.iota(jnp.int32, LANES) < _DEG_TAIL
    plsc.addupdate_scatter(hist_v, [tidx], ones, mask=tmask)
    pltpu.sync_copy(hist_v, out_hbm.at[wid])


def _make_scatter(width, npass):
    NG = NCHUNK // NBUF
    assert NG % 2 == 1 and NG >= 5

    @functools.partial(
        pl.kernel,
        out_type=jax.ShapeDtypeStruct((npass, N_PAD, 128), jnp.float32),
        mesh=_sc_mesh,
        scratch_types=[
            pltpu.VMEM((NCHUNK, CHUNK), jnp.int32),
            pltpu.VMEM((NCHUNK, CHUNK), jnp.int32),
            pltpu.VMEM((2 * NBUF, CHUNK, width), jnp.float32),
            pltpu.VMEM_SHARED((N_PAD, width), jnp.float32),
            pltpu.SemaphoreType.DMA((2 * NBUF,)),
            pltpu.SemaphoreType.DMA((2 * NBUF,)),
        ],
        compiler_params=pltpu.CompilerParams(needs_layout_passes=False,
                                             use_tc_tiling_on_sc=False),
    )
    def _scatter(hs_hbm, srcq_hbm, dst_hbm, zeros_hbm, out_hbm,
                 src_v, dst_v, rows_v, acc, gsem, ssem):
        c = lax.axis_index("c")
        s = lax.axis_index("s")
        slab = pl.ds(s * ROWS_PER_SUB, ROWS_PER_SUB)
        pltpu.sync_copy(dst_hbm.at[s], dst_v)

        def gather(k, slot):
            pltpu.async_copy(hs_hbm.at[src_v.at[k]], rows_v.at[slot],
                             gsem.at[slot])

        def do_group(g, bank):
            descs = []
            for b in range(NBUF):
                slot = bank * NBUF + b
                k = g * NBUF + b
                pltpu.make_async_copy(hs_hbm.at[src_v.at[k]], rows_v.at[slot],
                                      gsem.at[slot]).wait()
                descs.append(pltpu.async_copy(rows_v.at[slot],
                                              acc.at[dst_v.at[k]],
                                              ssem.at[slot], add=True))
            return descs

        def regather(g, bank, descs):
            for b in range(NBUF):
                descs[b].wait()
                gather(g * NBUF + b, bank * NBUF + b)

        @pl.loop(0, npass)
        def _pass(p):
            pltpu.sync_copy(srcq_hbm.at[p * NC + c, s], src_v)
            pltpu.sync_copy(zeros_hbm.at[slab], acc.at[slab])
            plsc.subcore_barrier()

            for slot in range(2 * NBUF):
                gather(slot, slot)

            @pl.loop(0, (NG - 3) // 2)
            def _g2(g2):
                for x in range(2):
                    g = 2 * g2 + x
                    regather(g + 2, x, do_group(g, x))

            d0 = do_group(NG - 3, (NG - 3) % 2)
            regather(NG - 1, (NG - 3) % 2, d0)
            tail = (do_group(NG - 2, (NG - 2) % 2)
                    + do_group(NG - 1, (NG - 1) % 2))
            for d in tail:
                d.wait()

            plsc.subcore_barrier()
            pltpu.sync_copy(acc.at[slab],
                            out_hbm.at[p].at[slab, pl.ds(c * width, width)])

    return _scatter


_scatter_hid = _make_scatter(HID // 4, 2)
_scatter_cls = _make_scatter(CLS // 2, 1)


_NBH = 1000
_GRID = NH // _NBH


def _dinv_of(dp_block):
    deg = jnp.sum(dp_block, axis=1) + 1.0
    return lax.rsqrt(deg)


def _mm1_body(xe_ref, xo_ref, w_ref, dpe_ref, dpo_ref, o4_ref):
    q = HID // 4
    dinve = _dinv_of(dpe_ref[...])
    dinvo = _dinv_of(dpo_ref[...])
    hse = jnp.dot(xe_ref[...], w_ref[...],
                  preferred_element_type=jnp.float32) * dinve[:, None]
    hso = jnp.dot(xo_ref[...], w_ref[...],
                  preferred_element_type=jnp.float32) * dinvo[:, None]
    for j in range(4):
        o4_ref[j] = jnp.concatenate(
            [hse[:, j * q:(j + 1) * q], hso[:, j * q:(j + 1) * q]], axis=1)


_mm1 = pl.pallas_call(
    _mm1_body,
    grid=(_GRID,),
    in_specs=[pl.BlockSpec((_NBH, F), lambda i: (i, 0)),
              pl.BlockSpec((_NBH, F), lambda i: (_GRID + i, 0)),
              pl.BlockSpec((F, HID), lambda i: (0, 0)),
              pl.BlockSpec((_NBH, NC * NS), lambda i: (i, 0)),
              pl.BlockSpec((_NBH, NC * NS), lambda i: (_GRID + i, 0))],
    out_specs=pl.BlockSpec((4, _NBH, 128), lambda i: (0, i, 0)),
    out_shape=jax.ShapeDtypeStruct((4, NH, 128), jnp.float32),
)


def _mid_body(ae_ref, ao_ref, t1_ref, dpe_ref, dpo_ref, b_ref, w_ref, o_ref):
    q = HID // 4
    dinve = _dinv_of(dpe_ref[...])
    dinvo = _dinv_of(dpo_ref[...])

    def half(a_ref, dinv, lo, hi):
        agg = jnp.concatenate([a_ref[0], a_ref[1]], axis=1)
        hs = jnp.concatenate([t1_ref[j][:, lo:hi] for j in range(4)], axis=1)
        z = (agg + hs) * dinv[:, None] + b_ref[...]
        a = jnp.maximum(z, 0.0)
        h2 = jnp.dot(a, w_ref[...], preferred_element_type=jnp.float32)
        return h2 * dinv[:, None]

    t2e = half(ae_ref, dinve, 0, q)
    t2o = half(ao_ref, dinvo, q, 2 * q)
    o_ref[...] = jnp.concatenate([t2e, t2o], axis=1)


_mid = pl.pallas_call(
    _mid_body,
    grid=(_GRID,),
    in_specs=[pl.BlockSpec((2, _NBH, 128), lambda i: (0, i, 0)),
              pl.BlockSpec((2, _NBH, 128), lambda i: (0, _GRID + i, 0)),
              pl.BlockSpec((4, _NBH, 128), lambda i: (0, i, 0)),
              pl.BlockSpec((_NBH, NC * NS), lambda i: (i, 0)),
              pl.BlockSpec((_NBH, NC * NS), lambda i: (_GRID + i, 0)),
              pl.BlockSpec((1, HID), lambda i: (0, 0)),
              pl.BlockSpec((HID, CLS), lambda i: (0, 0))],
    out_specs=pl.BlockSpec((_NBH, 128), lambda i: (i, 0)),
    out_shape=jax.ShapeDtypeStruct((NH, 128), jnp.float32),
)


def _final_body(ae_ref, ao_ref, t2_ref, dpe_ref, dpo_ref, b_ref, o_ref):
    dinve = _dinv_of(dpe_ref[...])
    dinvo = _dinv_of(dpo_ref[...])

    def half(a_ref, dinv, lo):
        z = ((a_ref[0][:, :CLS] + t2_ref[:, lo:lo + CLS])
             * dinv[:, None] + b_ref[...])
        m = jnp.max(z, axis=1, keepdims=True)
        lse = jnp.log(jnp.sum(jnp.exp(z - m), axis=1, keepdims=True)) + m
        return z - lse

    oute = half(ae_ref, dinve, 0)
    outo = half(ao_ref, dinvo, CLS)
    o_ref[0] = oute
    o_ref[1] = outo


_final = pl.pallas_call(
    _final_body,
    grid=(_GRID,),
    in_specs=[pl.BlockSpec((1, _NBH, 128), lambda i: (0, i, 0)),
              pl.BlockSpec((1, _NBH, 128), lambda i: (0, _GRID + i, 0)),
              pl.BlockSpec((_NBH, 128), lambda i: (i, 0)),
              pl.BlockSpec((_NBH, NC * NS), lambda i: (i, 0)),
              pl.BlockSpec((_NBH, NC * NS), lambda i: (_GRID + i, 0)),
              pl.BlockSpec((1, CLS), lambda i: (0, 0))],
    out_specs=pl.BlockSpec((2, _NBH, CLS), lambda i: (0, i, 0)),
    out_shape=jax.ShapeDtypeStruct((2, NH, CLS), jnp.float32),
)


def kernel(x, edge_index, W1, b1, W2, b2):
    src = edge_index[0].astype(jnp.int32)
    dst = edge_index[1].astype(jnp.int32)

    deg_parts = _deg_kernel(dst)
    dpt = deg_parts.T

    table1 = _mm1(x, x, W1, dpt, dpt)

    dst3 = dst.reshape(NS, NCHUNK, CHUNK)
    s2 = src.reshape(E // 128, 128)
    sh = (s2 >= NH).astype(jnp.int32)
    base1 = 2 * s2 - sh * (2 * NH - 1)
    srcq1 = (base1[None, :, :]
             + (jnp.arange(4, dtype=jnp.int32) * N)[:, None, None]
             ).reshape(4, NS, NCHUNK, CHUNK)
    srcq2 = (2 * base1[None, :, :]
             + jnp.arange(2, dtype=jnp.int32)[:, None, None]
             ).reshape(2, NS, NCHUNK, CHUNK)

    agg1 = _scatter_hid(table1.reshape(4 * N, HID // 4), srcq1, dst3,
                        jnp.zeros((N_PAD, HID // 4), jnp.float32))
    table2 = _mid(agg1, agg1, table1, dpt, dpt, b1.reshape(1, HID), W2)
    agg2 = _scatter_cls(table2.reshape(2 * N, CLS // 2), srcq2, dst3,
                        jnp.zeros((N_PAD, CLS // 2), jnp.float32))
    outp = _final(agg2, agg2, table2, dpt, dpt, b2.reshape(1, CLS))
    return outp.reshape(N, CLS)
